# baseline (speedup 1.0000x reference)
import jax
import jax.numpy as jnp
from jax.experimental import pallas as pl
from jax.experimental.pallas import tpu as pltpu

_RADIUS = 5
_H = 128
_W = 128
_IN = _H - 2 * _RADIUS

_DELTAS = [
    (dx, dy)
    for dx in range(-_RADIUS, _RADIUS + 1)
    for dy in range(-_RADIUS, _RADIUS + 1)
    if dx * dx + dy * dy < _RADIUS * _RADIUS and dx + dy != 0
]


def _loss_kernel(df0_ref, df1_ref, tg_ref, out_ref, tp_ref):
    r = _RADIUS
    tp_ref[...] = jnp.where(tg_ref[...] > 0, jnp.float32(1.0), jnp.float32(0.0))

    accf = jnp.zeros((_IN, _IN), jnp.float32)
    accb = jnp.zeros((_IN, _IN), jnp.float32)
    accc = jnp.zeros((_IN, _IN), jnp.float32)
    for b in range(tg_ref.shape[0]):
        f0c = df0_ref[b, r:r + _IN, r:r + _IN]
        f1c = df1_ref[b, r:r + _IN, r:r + _IN]
        tcf = tp_ref[b, r:r + _IN, r:r + _IN]
        for dx, dy in _DELTAS:
            ys = r + dy
            xs = r + dx
            d0 = df0_ref[b, ys:ys + _IN, xs:xs + _IN] - f0c
            d1 = df1_ref[b, ys:ys + _IN, xs:xs + _IN] - f1c
            fgf = tcf * tp_ref[b, ys:ys + _IN, xs:xs + _IN]
            ab = jnp.abs(d0 - jnp.float32(dy)) + jnp.abs(d1 - jnp.float32(dx))
            s = d0 + d1
            accf = accf + fgf * ab
            accb = accb + (s - fgf * s)
            accc = accc + fgf

    fg_sum = jnp.sum(accf)
    bg_sum = jnp.sum(accb)
    fg_cnt = jnp.sum(accc)
    total = jnp.float32(len(_DELTAS) * _IN * _IN * tg_ref.shape[0])
    bg_cnt = total - fg_cnt
    loss = (fg_sum / jnp.maximum(fg_cnt, 1.0)
            + bg_sum / jnp.maximum(bg_cnt, 1.0))
    out_ref[:, :] = loss[None, None]


def kernel(df, bd, targets):
    del bd
    B = df.shape[0]
    df0 = df[:, 0]
    df1 = df[:, 1]
    out = pl.pallas_call(
        _loss_kernel,
        out_shape=jax.ShapeDtypeStruct((1, 1), jnp.float32),
        scratch_shapes=[pltpu.VMEM((B, _H, _W), jnp.float32)],
    )(df0, df1, targets)
    return out[0, 0]


# speedup vs baseline: 2.0669x; 2.0669x over previous
import jax
import jax.numpy as jnp
from jax.experimental import pallas as pl
from jax.experimental.pallas import tpu as pltpu

_RADIUS = 5
_H = 128
_W = 128
_IN = _H - 2 * _RADIUS

_DELTAS = [
    (dx, dy)
    for dx in range(-_RADIUS, _RADIUS + 1)
    for dy in range(-_RADIUS, _RADIUS + 1)
    if dx * dx + dy * dy < _RADIUS * _RADIUS and dx + dy != 0
]


def _loss_kernel(df_ref, tg_ref, out_ref, tp_ref):
    r = _RADIUS
    tp_ref[...] = jnp.where(tg_ref[...] > 0, jnp.float32(1.0), jnp.float32(0.0))

    f0c = df_ref[:, 0, r:r + _IN, r:r + _IN]
    f1c = df_ref[:, 1, r:r + _IN, r:r + _IN]
    tcf = tp_ref[:, r:r + _IN, r:r + _IN]

    accf = jnp.zeros((_IN, _IN), jnp.float32)
    accb = jnp.zeros((_IN, _IN), jnp.float32)
    accc = jnp.zeros((_IN, _IN), jnp.float32)
    for dx, dy in _DELTAS:
        ys = r + dy
        xs = r + dx
        d0 = df_ref[:, 0, ys:ys + _IN, xs:xs + _IN] - f0c
        d1 = df_ref[:, 1, ys:ys + _IN, xs:xs + _IN] - f1c
        fgf = tcf * tp_ref[:, ys:ys + _IN, xs:xs + _IN]
        ab = jnp.abs(d0 - jnp.float32(dy)) + jnp.abs(d1 - jnp.float32(dx))
        s = d0 + d1
        accf = accf + jnp.sum(fgf * ab, axis=0)
        accb = accb + jnp.sum(s - fgf * s, axis=0)
        accc = accc + jnp.sum(fgf, axis=0)

    fg_sum = jnp.sum(accf)
    bg_sum = jnp.sum(accb)
    fg_cnt = jnp.sum(accc)
    total = jnp.float32(len(_DELTAS) * _IN * _IN * tg_ref.shape[0])
    bg_cnt = total - fg_cnt
    loss = (fg_sum / jnp.maximum(fg_cnt, 1.0)
            + bg_sum / jnp.maximum(bg_cnt, 1.0))
    out_ref[:, :] = loss[None, None]


def kernel(df, bd, targets):
    del bd
    B = df.shape[0]
    out = pl.pallas_call(
        _loss_kernel,
        out_shape=jax.ShapeDtypeStruct((1, 1), jnp.float32),
        scratch_shapes=[pltpu.VMEM((B, _H, _W), jnp.float32)],
    )(df, targets)
    return out[0, 0]


# speedup vs baseline: 2.0805x; 1.0066x over previous
import jax
import jax.numpy as jnp
from jax.experimental import pallas as pl
from jax.experimental.pallas import tpu as pltpu

_RADIUS = 5
_H = 128
_W = 128
_IN = _H - 2 * _RADIUS

_DELTAS = [
    (dx, dy)
    for dx in range(-_RADIUS, _RADIUS + 1)
    for dy in range(-_RADIUS, _RADIUS + 1)
    if dx * dx + dy * dy < _RADIUS * _RADIUS and dx + dy != 0
]


def _loss_kernel(df_ref, tg_ref, out_ref, tp_ref, r0_ref, r1_ref, rt_ref):
    r = _RADIUS
    tp_ref[...] = jnp.where(tg_ref[...] > 0, jnp.float32(1.0), jnp.float32(0.0))

    for j in range(2 * _RADIUS - 1):
        ys = j + 1
        r0_ref[j] = df_ref[:, 0, ys:ys + _IN, :]
        r1_ref[j] = df_ref[:, 1, ys:ys + _IN, :]
        rt_ref[j] = tp_ref[:, ys:ys + _IN, :]

    f0c = r0_ref[r - 1, :, :, r:r + _IN]
    f1c = r1_ref[r - 1, :, :, r:r + _IN]
    tcf = rt_ref[r - 1, :, :, r:r + _IN]

    accf = jnp.zeros((_IN, _IN), jnp.float32)
    accb = jnp.zeros((_IN, _IN), jnp.float32)
    accc = jnp.zeros((_IN, _IN), jnp.float32)
    for dx, dy in _DELTAS:
        j = r + dy - 1
        xs = r + dx
        d0 = r0_ref[j, :, :, xs:xs + _IN] - f0c
        d1 = r1_ref[j, :, :, xs:xs + _IN] - f1c
        fgf = tcf * rt_ref[j, :, :, xs:xs + _IN]
        ab = jnp.abs(d0 - jnp.float32(dy)) + jnp.abs(d1 - jnp.float32(dx))
        s = d0 + d1
        accf = accf + jnp.sum(fgf * ab, axis=0)
        accb = accb + jnp.sum(s - fgf * s, axis=0)
        accc = accc + jnp.sum(fgf, axis=0)

    fg_sum = jnp.sum(accf)
    bg_sum = jnp.sum(accb)
    fg_cnt = jnp.sum(accc)
    total = jnp.float32(len(_DELTAS) * _IN * _IN * tg_ref.shape[0])
    bg_cnt = total - fg_cnt
    loss = (fg_sum / jnp.maximum(fg_cnt, 1.0)
            + bg_sum / jnp.maximum(bg_cnt, 1.0))
    out_ref[:, :] = loss[None, None]


def kernel(df, bd, targets):
    del bd
    B = df.shape[0]
    out = pl.pallas_call(
        _loss_kernel,
        out_shape=jax.ShapeDtypeStruct((1, 1), jnp.float32),
        scratch_shapes=[
            pltpu.VMEM((B, _H, _W), jnp.float32),
            pltpu.VMEM((2 * _RADIUS - 1, B, _IN, _W), jnp.float32),
            pltpu.VMEM((2 * _RADIUS - 1, B, _IN, _W), jnp.float32),
            pltpu.VMEM((2 * _RADIUS - 1, B, _IN, _W), jnp.float32),
        ],
    )(df, targets)
    return out[0, 0]
